# baseline (speedup 1.0000x reference)
import functools

import jax
import jax.numpy as jnp
from jax import lax
from jax.experimental import pallas as pl
from jax.experimental.pallas import tpu as pltpu
from jax.experimental.pallas import tpu_sc as plsc

_EMB_DIM = 64
_NB = 16384
_NJ = 20
_B = _NB * _NJ

_info = plsc.get_sparse_core_info()
_NC, _NS = _info.num_cores, _info.num_subcores
_NW = _NC * _NS
_B_PER_W = _B // _NW

_CH = 512
_NSLOT = 3
_N_CH = _B_PER_W // _CH
_N_IT = (_N_CH - 2) // 3


def _gather_body(x_hbm, t_hbm, out_hbm, idx_v, rows, gsem, wsem):
    wid = lax.axis_index("s") * _NC + lax.axis_index("c")
    base = wid * _B_PER_W

    pltpu.sync_copy(x_hbm.at[pl.ds(base, _B_PER_W)], idx_v)

    def idx_slice(i):
        return idx_v.at[pl.ds(pl.multiple_of(i * _CH, 8), _CH)]

    def fire_g(i, s):
        pltpu.async_copy(t_hbm.at[idx_slice(i)], rows.at[s], gsem.at[s])

    def wait_g(i, s):
        pltpu.make_async_copy(
            t_hbm.at[idx_slice(i)], rows.at[s], gsem.at[s]).wait()

    def fire_w(i, s):
        off = pl.multiple_of(base + i * _CH, 8)
        pltpu.async_copy(rows.at[s], out_hbm.at[pl.ds(off, _CH)], wsem.at[s])

    def drain_w(s):
        pltpu.make_async_copy(
            rows.at[s], out_hbm.at[pl.ds(0, _CH)], wsem.at[s]).wait()

    fire_g(0, 0)
    fire_g(1, 1)

    def iteration(t, carry):
        for j in range(3):
            i = 3 * t + 1 + j
            wait_g(i - 1, j)
            fire_w(i - 1, j)
            if j == 0:
                @pl.when(t > 0)
                def _():
                    drain_w((j + 2) % 3)
            else:
                drain_w((j + 2) % 3)
            fire_g(i + 1, (j + 2) % 3)
        return carry

    lax.fori_loop(0, _N_IT, iteration, 0)

    p = _N_CH - 2
    wait_g(p, p % _NSLOT)
    fire_w(p, p % _NSLOT)
    drain_w((p - 1) % _NSLOT)
    last = _N_CH - 1
    wait_g(last, last % _NSLOT)
    fire_w(last, last % _NSLOT)
    drain_w(p % _NSLOT)
    drain_w(last % _NSLOT)


def _make_gather():
    mesh = plsc.VectorSubcoreMesh(core_axis_name="c", subcore_axis_name="s")
    out_type = jax.ShapeDtypeStruct((_B, _EMB_DIM), jnp.float32)
    scratch = [
        pltpu.VMEM((_B_PER_W,), jnp.int32),
        pltpu.VMEM((_NSLOT, _CH, _EMB_DIM), jnp.float32),
        pltpu.SemaphoreType.DMA((_NSLOT,)),
        pltpu.SemaphoreType.DMA((_NSLOT,)),
    ]
    return functools.partial(
        pl.kernel,
        out_type=out_type,
        mesh=mesh,
        scratch_types=scratch,
        compiler_params=pltpu.CompilerParams(use_tc_tiling_on_sc=False),
    )(_gather_body)


def kernel(x, L, R):
    x_flat = x.reshape(-1)
    g = _make_gather()
    l_flat = g(x_flat, L)
    r_flat = g(x_flat, R)
    shape = x.shape + (_EMB_DIM,)
    return (l_flat.reshape(shape), r_flat.reshape(shape))


# speedup vs baseline: 1.0014x; 1.0014x over previous
import functools

import jax
import jax.numpy as jnp
from jax import lax
from jax.experimental import pallas as pl
from jax.experimental.pallas import tpu as pltpu
from jax.experimental.pallas import tpu_sc as plsc

_EMB_DIM = 64
_NB = 16384
_NJ = 20
_B = _NB * _NJ

_info = plsc.get_sparse_core_info()
_NC, _NS = _info.num_cores, _info.num_subcores
_NW = _NC * _NS
_B_PER_W = _B // _NW

_CH = 256
_NSLOT = 3
_N_CH = _B_PER_W // _CH
_N_IT = (_N_CH - 1) // 3


def _gather_body(x_hbm, t_hbm, out_hbm, idx_v, rows, gsem, wsem):
    wid = lax.axis_index("s") * _NC + lax.axis_index("c")
    base = wid * _B_PER_W

    pltpu.sync_copy(x_hbm.at[pl.ds(base, _B_PER_W)], idx_v)

    def idx_slice(i):
        return idx_v.at[pl.ds(pl.multiple_of(i * _CH, 8), _CH)]

    def fire_g(i, s):
        pltpu.async_copy(t_hbm.at[idx_slice(i)], rows.at[s], gsem.at[s])

    def wait_g(i, s):
        pltpu.make_async_copy(
            t_hbm.at[idx_slice(i)], rows.at[s], gsem.at[s]).wait()

    def fire_w(i, s):
        off = pl.multiple_of(base + i * _CH, 8)
        pltpu.async_copy(rows.at[s], out_hbm.at[pl.ds(off, _CH)], wsem.at[s])

    def drain_w(s):
        pltpu.make_async_copy(
            rows.at[s], out_hbm.at[pl.ds(0, _CH)], wsem.at[s]).wait()

    fire_g(0, 0)
    fire_g(1, 1)

    def iteration(t, carry):
        for j in range(3):
            i = 3 * t + 1 + j
            wait_g(i - 1, j)
            fire_w(i - 1, j)
            if j == 0:
                @pl.when(t > 0)
                def _():
                    drain_w((j + 2) % 3)
            else:
                drain_w((j + 2) % 3)
            if j == 2:
                @pl.when(t < _N_IT - 1)
                def _():
                    fire_g(i + 1, (j + 2) % 3)
            else:
                fire_g(i + 1, (j + 2) % 3)
        return carry

    lax.fori_loop(0, _N_IT, iteration, 0)

    drain_w(2)
    last = _N_CH - 1
    wait_g(last, last % _NSLOT)
    fire_w(last, last % _NSLOT)
    drain_w(last % _NSLOT)


def _make_gather():
    mesh = plsc.VectorSubcoreMesh(core_axis_name="c", subcore_axis_name="s")
    out_type = jax.ShapeDtypeStruct((_B, _EMB_DIM), jnp.float32)
    scratch = [
        pltpu.VMEM((_B_PER_W,), jnp.int32),
        pltpu.VMEM((_NSLOT, _CH, _EMB_DIM), jnp.float32),
        pltpu.SemaphoreType.DMA((_NSLOT,)),
        pltpu.SemaphoreType.DMA((_NSLOT,)),
    ]
    return functools.partial(
        pl.kernel,
        out_type=out_type,
        mesh=mesh,
        scratch_types=scratch,
        compiler_params=pltpu.CompilerParams(use_tc_tiling_on_sc=False),
    )(_gather_body)


def kernel(x, L, R):
    x_flat = x.reshape(-1)
    g = _make_gather()
    l_flat = g(x_flat, L)
    r_flat = g(x_flat, R)
    shape = x.shape + (_EMB_DIM,)
    return (l_flat.reshape(shape), r_flat.reshape(shape))
